# baseline (speedup 1.0000x reference)
import functools

import jax
import jax.numpy as jnp
from jax import lax
from jax.experimental import pallas as pl
from jax.experimental.pallas import tpu as pltpu
from jax.experimental.pallas import tpu_sc as plsc

_NC = 2
_NS = 16
_NW = _NC * _NS
_G = 128


def _sc_degree(dstb, ones8, zrows8, NP, GPW):
    rpt = NP // _NS
    mesh = plsc.VectorSubcoreMesh(core_axis_name="c", subcore_axis_name="s")

    @functools.partial(
        pl.kernel,
        out_type=jax.ShapeDtypeStruct((_NC, NP, 8), jnp.float32),
        mesh=mesh,
        scratch_types=[
            pltpu.VMEM((GPW, _G), jnp.int32),
            pltpu.VMEM((_G, 8), jnp.float32),
            pltpu.VMEM_SHARED((NP, 8), jnp.float32),
        ],
        compiler_params=pltpu.CompilerParams(use_tc_tiling_on_sc=False),
    )
    def deg_kernel(dstb_hbm, ones_hbm, z_hbm, out_hbm, dst_v, ones_v, acc):
        c = lax.axis_index("c")
        s = lax.axis_index("s")
        wid = c * _NS + s
        base = s * rpt
        pltpu.sync_copy(z_hbm, acc.at[pl.ds(base, rpt)])
        pltpu.sync_copy(ones_hbm, ones_v)
        pltpu.sync_copy(dstb_hbm.at[wid], dst_v)
        plsc.subcore_barrier()

        @pl.loop(0, GPW)
        def _(g):
            pltpu.sync_copy(ones_v, acc.at[dst_v.at[g]], add=True)

        plsc.subcore_barrier()
        pltpu.sync_copy(acc.at[pl.ds(base, rpt)],
                        out_hbm.at[c, pl.ds(base, rpt)])

    return deg_kernel(dstb, ones8, zrows8)


def _sc_hop(u, srcb, dstb, zrows, NP, CP, GPW):
    rpt = NP // _NS
    mesh = plsc.VectorSubcoreMesh(core_axis_name="c", subcore_axis_name="s")

    @functools.partial(
        pl.kernel,
        out_type=jax.ShapeDtypeStruct((_NC, NP, CP), jnp.float32),
        mesh=mesh,
        scratch_types=[
            pltpu.VMEM((GPW, _G), jnp.int32),
            pltpu.VMEM((GPW, _G), jnp.int32),
            pltpu.VMEM((_G, CP), jnp.float32),
            pltpu.VMEM_SHARED((NP, CP), jnp.float32),
            pltpu.SemaphoreType.DMA,
        ],
        compiler_params=pltpu.CompilerParams(use_tc_tiling_on_sc=False),
    )
    def hop_kernel(u_hbm, srcb_hbm, dstb_hbm, z_hbm, out_hbm,
                   src_v, dst_v, rows_v, acc, gs0):
        c = lax.axis_index("c")
        s = lax.axis_index("s")
        wid = c * _NS + s
        base = s * rpt
        pltpu.sync_copy(z_hbm, acc.at[pl.ds(base, rpt)])
        pltpu.sync_copy(srcb_hbm.at[wid], src_v)
        pltpu.sync_copy(dstb_hbm.at[wid], dst_v)
        plsc.subcore_barrier()

        @pl.loop(0, GPW)
        def _(g):
            pltpu.async_copy(u_hbm.at[src_v.at[g]], rows_v, gs0).wait()
            pltpu.sync_copy(rows_v, acc.at[dst_v.at[g]], add=True)
        plsc.subcore_barrier()
        pltpu.sync_copy(acc.at[pl.ds(base, rpt)],
                        out_hbm.at[c, pl.ds(base, rpt)])

    return hop_kernel(u, srcb, dstb, zrows)


def _tc_scale0(degp, xp, W48, NP, D, CP, n_real, BR):
    def body(d_ref, x_ref, w_ref, u_ref, dv_ref):
        i = pl.program_id(0)
        z = jnp.dot(x_ref[...], w_ref[...],
                    preferred_element_type=jnp.float32)
        deg = d_ref[0] + d_ref[1] + 1.0
        row = lax.broadcasted_iota(jnp.int32, (BR, 8), 0) + i * BR
        dinv = jnp.where(row < n_real, lax.rsqrt(deg), 0.0)
        dv_ref[...] = dinv
        u_ref[...] = z * dinv[:, 0:1]

    return pl.pallas_call(
        body,
        grid=(NP // BR,),
        in_specs=[pl.BlockSpec((2, BR, 8), lambda i: (0, i, 0)),
                  pl.BlockSpec((BR, D), lambda i: (i, 0)),
                  pl.BlockSpec((D, CP), lambda i: (0, 0))],
        out_specs=[pl.BlockSpec((BR, CP), lambda i: (i, 0)),
                   pl.BlockSpec((BR, 8), lambda i: (i, 0))],
        out_shape=[jax.ShapeDtypeStruct((NP, CP), jnp.float32),
                   jax.ShapeDtypeStruct((NP, 8), jnp.float32)],
    )(degp, xp, W48)


def _tc_scale1(hp, u0, dinv8, NP, CP, BR):
    def body(h_ref, u_ref, d_ref, o_ref):
        d = d_ref[:, 0:1]
        o_ref[...] = (h_ref[0] + h_ref[1] + u_ref[...]) * (d * d)

    return pl.pallas_call(
        body,
        grid=(NP // BR,),
        in_specs=[pl.BlockSpec((2, BR, CP), lambda i: (0, i, 0)),
                  pl.BlockSpec((BR, CP), lambda i: (i, 0)),
                  pl.BlockSpec((BR, 8), lambda i: (i, 0))],
        out_specs=pl.BlockSpec((BR, CP), lambda i: (i, 0)),
        out_shape=jax.ShapeDtypeStruct((NP, CP), jnp.float32),
    )(hp, u0, dinv8)


def _tc_out(qp, u1, dinv8, b48, NP, N, C, CP, BR):
    def body(q_ref, u_ref, d_ref, b_ref, o_ref):
        t = (q_ref[0] + q_ref[1] + u_ref[...]) * d_ref[:, 0:1]
        o40 = t[:, :C] + b_ref[0:1, :C]
        mx = jnp.max(o40, axis=1, keepdims=True)
        ex = jnp.exp(o40 - mx)
        sm = jnp.sum(ex, axis=1, keepdims=True)
        o_ref[...] = o40 - mx - jnp.log(sm)

    return pl.pallas_call(
        body,
        grid=(N // BR,),
        in_specs=[pl.BlockSpec((2, BR, CP), lambda i: (0, i, 0)),
                  pl.BlockSpec((BR, CP), lambda i: (i, 0)),
                  pl.BlockSpec((BR, 8), lambda i: (i, 0)),
                  pl.BlockSpec((8, CP), lambda i: (0, 0))],
        out_specs=pl.BlockSpec((BR, C), lambda i: (i, 0)),
        out_shape=jax.ShapeDtypeStruct((N, C), jnp.float32),
    )(qp, u1, dinv8, b48)


def kernel(x, edge_index, W, b):
    N, D = x.shape
    C = W.shape[1]
    E = edge_index.shape[1]
    CP = 48
    NP = -(-(N + 1) // (_NS * 8)) * (_NS * 8)
    GPW = -(-E // (_NW * _G))
    GPW += GPW % 2
    EP = GPW * _NW * _G

    src = edge_index[0]
    dst = edge_index[1]
    fill = jnp.full((EP - E,), N, jnp.int32)
    srcb = jnp.concatenate([src, fill]).reshape(_NW, GPW, _G)
    dstb = jnp.concatenate([dst, fill]).reshape(_NW, GPW, _G)

    xp = jnp.pad(x, ((0, NP - N), (0, 0)))
    W48 = jnp.pad(W, ((0, 0), (0, CP - C)))
    b48 = jnp.broadcast_to(jnp.pad(b, (0, CP - C))[None, :], (8, CP))
    ones8 = jnp.ones((_G, 8), jnp.float32)
    z8 = jnp.zeros((NP // _NS, 8), jnp.float32)
    zCP = jnp.zeros((NP // _NS, CP), jnp.float32)

    BR = NP // 8
    degp = _sc_degree(dstb, ones8, z8, NP, GPW)
    u0, dinv8 = _tc_scale0(degp, xp, W48, NP, D, CP, N, BR)
    hp = _sc_hop(u0, srcb, dstb, zCP, NP, CP, GPW)
    u1 = _tc_scale1(hp, u0, dinv8, NP, CP, BR)
    qp = _sc_hop(u1, srcb, dstb, zCP, NP, CP, GPW)
    return _tc_out(qp, u1, dinv8, b48, NP, N, C, CP, 400)


# speedup vs baseline: 1.5847x; 1.5847x over previous
import functools

import jax
import jax.numpy as jnp
from jax import lax
from jax.experimental import pallas as pl
from jax.experimental.pallas import tpu as pltpu
from jax.experimental.pallas import tpu_sc as plsc

_NC = 2
_NS = 16
_NW = _NC * _NS
_G = 128


def _sc_degree(dstb, ones8, zrows8, NP, GPW):
    rpt = NP // _NS
    mesh = plsc.VectorSubcoreMesh(core_axis_name="c", subcore_axis_name="s")

    @functools.partial(
        pl.kernel,
        out_type=jax.ShapeDtypeStruct((_NC, NP, 8), jnp.float32),
        mesh=mesh,
        scratch_types=[
            pltpu.VMEM((GPW, _G), jnp.int32),
            pltpu.VMEM((_G, 8), jnp.float32),
            pltpu.VMEM_SHARED((NP, 8), jnp.float32),
        ],
        compiler_params=pltpu.CompilerParams(use_tc_tiling_on_sc=False),
    )
    def deg_kernel(dstb_hbm, ones_hbm, z_hbm, out_hbm, dst_v, ones_v, acc):
        c = lax.axis_index("c")
        s = lax.axis_index("s")
        wid = c * _NS + s
        base = s * rpt
        pltpu.sync_copy(z_hbm, acc.at[pl.ds(base, rpt)])
        pltpu.sync_copy(ones_hbm, ones_v)
        pltpu.sync_copy(dstb_hbm.at[wid], dst_v)
        plsc.subcore_barrier()

        @pl.loop(0, GPW)
        def _(g):
            pltpu.sync_copy(ones_v, acc.at[dst_v.at[g]], add=True)

        plsc.subcore_barrier()
        pltpu.sync_copy(acc.at[pl.ds(base, rpt)],
                        out_hbm.at[c, pl.ds(base, rpt)])

    return deg_kernel(dstb, ones8, zrows8)


def _sc_hop(u, srcb, dstb, zrows, NP, CP, GPW):
    rpt = NP // _NS
    mesh = plsc.VectorSubcoreMesh(core_axis_name="c", subcore_axis_name="s")

    @functools.partial(
        pl.kernel,
        out_type=jax.ShapeDtypeStruct((_NC, NP, CP), jnp.float32),
        mesh=mesh,
        scratch_types=[
            pltpu.VMEM((GPW, _G), jnp.int32),
            pltpu.VMEM((GPW, _G), jnp.int32),
            pltpu.VMEM((_G, CP), jnp.float32),
            pltpu.VMEM_SHARED((NP, CP), jnp.float32),
            pltpu.SemaphoreType.DMA,
        ],
        compiler_params=pltpu.CompilerParams(use_tc_tiling_on_sc=False),
    )
    def hop_kernel(u_hbm, srcb_hbm, dstb_hbm, z_hbm, out_hbm,
                   src_v, dst_v, rows_v, acc, gs0):
        c = lax.axis_index("c")
        s = lax.axis_index("s")
        wid = c * _NS + s
        base = s * rpt
        pltpu.sync_copy(z_hbm, acc.at[pl.ds(base, rpt)])
        pltpu.sync_copy(srcb_hbm.at[wid], src_v)
        pltpu.sync_copy(dstb_hbm.at[wid], dst_v)
        plsc.subcore_barrier()

        @pl.loop(0, GPW)
        def _(g):
            pltpu.async_copy(u_hbm.at[src_v.at[g]], rows_v, gs0).wait()
            pltpu.sync_copy(rows_v, acc.at[dst_v.at[g]], add=True)
        plsc.subcore_barrier()
        pltpu.sync_copy(acc.at[pl.ds(base, rpt)],
                        out_hbm.at[c, pl.ds(base, rpt)])

    return hop_kernel(u, srcb, dstb, zrows)


def _tc_scale0(degp, xp, W48, NP, D, CP, n_real, BR):
    def body(d_ref, x_ref, w_ref, u_ref, dv_ref):
        i = pl.program_id(0)
        z = jnp.dot(x_ref[...], w_ref[...],
                    preferred_element_type=jnp.float32)
        deg = d_ref[0] + d_ref[1] + 1.0
        row = lax.broadcasted_iota(jnp.int32, (BR, 8), 0) + i * BR
        dinv = jnp.where(row < n_real, lax.rsqrt(deg), 0.0)
        dv_ref[...] = dinv
        u_ref[...] = z * dinv[:, 0:1]

    return pl.pallas_call(
        body,
        grid=(NP // BR,),
        in_specs=[pl.BlockSpec((2, BR, 8), lambda i: (0, i, 0)),
                  pl.BlockSpec((BR, D), lambda i: (i, 0)),
                  pl.BlockSpec((D, CP), lambda i: (0, 0))],
        out_specs=[pl.BlockSpec((BR, CP), lambda i: (i, 0)),
                   pl.BlockSpec((BR, 8), lambda i: (i, 0))],
        out_shape=[jax.ShapeDtypeStruct((NP, CP), jnp.float32),
                   jax.ShapeDtypeStruct((NP, 8), jnp.float32)],
    )(degp, xp, W48)


def _tc_scale1(hp, u0, dinv8, NP, CP, BR):
    def body(h_ref, u_ref, d_ref, o_ref):
        d = d_ref[:, 0:1]
        o_ref[...] = (h_ref[0] + h_ref[1] + u_ref[...]) * (d * d)

    return pl.pallas_call(
        body,
        grid=(NP // BR,),
        in_specs=[pl.BlockSpec((2, BR, CP), lambda i: (0, i, 0)),
                  pl.BlockSpec((BR, CP), lambda i: (i, 0)),
                  pl.BlockSpec((BR, 8), lambda i: (i, 0))],
        out_specs=pl.BlockSpec((BR, CP), lambda i: (i, 0)),
        out_shape=jax.ShapeDtypeStruct((NP, CP), jnp.float32),
    )(hp, u0, dinv8)


def _tc_out(qp, u1, dinv8, b48, NP, N, C, CP, BR):
    def body(q_ref, u_ref, d_ref, b_ref, o_ref):
        t = (q_ref[0] + q_ref[1] + u_ref[...]) * d_ref[:, 0:1]
        o40 = t[:, :C] + b_ref[0:1, :C]
        mx = jnp.max(o40, axis=1, keepdims=True)
        ex = jnp.exp(o40 - mx)
        sm = jnp.sum(ex, axis=1, keepdims=True)
        o_ref[...] = o40 - mx - jnp.log(sm)

    return pl.pallas_call(
        body,
        grid=(N // BR,),
        in_specs=[pl.BlockSpec((2, BR, CP), lambda i: (0, i, 0)),
                  pl.BlockSpec((BR, CP), lambda i: (i, 0)),
                  pl.BlockSpec((BR, 8), lambda i: (i, 0)),
                  pl.BlockSpec((8, CP), lambda i: (0, 0))],
        out_specs=pl.BlockSpec((BR, C), lambda i: (i, 0)),
        out_shape=jax.ShapeDtypeStruct((N, C), jnp.float32),
    )(qp, u1, dinv8, b48)


def kernel(x, edge_index, W, b):
    N, D = x.shape
    C = W.shape[1]
    E = edge_index.shape[1]
    CP = 48
    NP = -(-(N + 1) // (_NS * 8)) * (_NS * 8)
    GPW = -(-E // (_NW * _G))
    GPW += GPW % 2
    EP = GPW * _NW * _G

    src = edge_index[0]
    dst = edge_index[1]
    fill = N + jnp.arange(EP - E, dtype=jnp.int32) % (NP - N)
    srcb = jnp.concatenate([src, fill]).reshape(_NW, GPW, _G)
    dstb = jnp.concatenate([dst, fill]).reshape(_NW, GPW, _G)

    xp = jnp.pad(x, ((0, NP - N), (0, 0)))
    W48 = jnp.pad(W, ((0, 0), (0, CP - C)))
    b48 = jnp.broadcast_to(jnp.pad(b, (0, CP - C))[None, :], (8, CP))
    ones8 = jnp.ones((_G, 8), jnp.float32)
    z8 = jnp.zeros((NP // _NS, 8), jnp.float32)
    zCP = jnp.zeros((NP // _NS, CP), jnp.float32)

    BR = NP // 8
    degp = _sc_degree(dstb, ones8, z8, NP, GPW)
    u0, dinv8 = _tc_scale0(degp, xp, W48, NP, D, CP, N, BR)
    hp = _sc_hop(u0, srcb, dstb, zCP, NP, CP, GPW)
    u1 = _tc_scale1(hp, u0, dinv8, NP, CP, BR)
    qp = _sc_hop(u1, srcb, dstb, zCP, NP, CP, GPW)
    return _tc_out(qp, u1, dinv8, b48, NP, N, C, CP, 400)


# speedup vs baseline: 1.8036x; 1.1382x over previous
import functools

import jax
import jax.numpy as jnp
from jax import lax
from jax.experimental import pallas as pl
from jax.experimental.pallas import tpu as pltpu
from jax.experimental.pallas import tpu_sc as plsc

_NC = 2
_NS = 16
_NW = _NC * _NS
_G = 128


def _sc_degree(dstb, ones8, zrows8, NP, GPW):
    rpt = NP // _NS
    mesh = plsc.VectorSubcoreMesh(core_axis_name="c", subcore_axis_name="s")

    @functools.partial(
        pl.kernel,
        out_type=jax.ShapeDtypeStruct((_NC, NP, 8), jnp.float32),
        mesh=mesh,
        scratch_types=[
            pltpu.VMEM((GPW, _G), jnp.int32),
            pltpu.VMEM((_G, 8), jnp.float32),
            pltpu.VMEM_SHARED((NP, 8), jnp.float32),
        ],
        compiler_params=pltpu.CompilerParams(use_tc_tiling_on_sc=False),
    )
    def deg_kernel(dstb_hbm, ones_hbm, z_hbm, out_hbm, dst_v, ones_v, acc):
        c = lax.axis_index("c")
        s = lax.axis_index("s")
        wid = c * _NS + s
        base = s * rpt
        pltpu.sync_copy(z_hbm, acc.at[pl.ds(base, rpt)])
        pltpu.sync_copy(ones_hbm, ones_v)
        pltpu.sync_copy(dstb_hbm.at[wid], dst_v)
        plsc.subcore_barrier()

        @pl.loop(0, GPW)
        def _(g):
            pltpu.sync_copy(ones_v, acc.at[dst_v.at[g]], add=True)

        plsc.subcore_barrier()
        pltpu.sync_copy(acc.at[pl.ds(base, rpt)],
                        out_hbm.at[c, pl.ds(base, rpt)])

    return deg_kernel(dstb, ones8, zrows8)


def _sc_hop(u, srcb, dstb, zrows, NP, CP, GPW):
    rpt = NP // _NS
    mesh = plsc.VectorSubcoreMesh(core_axis_name="c", subcore_axis_name="s")

    @functools.partial(
        pl.kernel,
        out_type=jax.ShapeDtypeStruct((_NC, NP, CP), jnp.float32),
        mesh=mesh,
        scratch_types=[
            pltpu.VMEM((GPW, _G), jnp.int32),
            pltpu.VMEM((GPW, _G), jnp.int32),
            pltpu.VMEM((2, _G, CP), jnp.float32),
            pltpu.VMEM_SHARED((NP, CP), jnp.float32),
            pltpu.SemaphoreType.DMA,
            pltpu.SemaphoreType.DMA,
            pltpu.SemaphoreType.DMA,
            pltpu.SemaphoreType.DMA,
        ],
        compiler_params=pltpu.CompilerParams(use_tc_tiling_on_sc=False),
    )
    def hop_kernel(u_hbm, srcb_hbm, dstb_hbm, z_hbm, out_hbm,
                   src_v, dst_v, rows_v, acc, gs0, gs1, ss0, ss1):
        c = lax.axis_index("c")
        s = lax.axis_index("s")
        wid = c * _NS + s
        base = s * rpt
        pltpu.sync_copy(z_hbm, acc.at[pl.ds(base, rpt)])
        pltpu.sync_copy(srcb_hbm.at[wid], src_v)
        pltpu.sync_copy(dstb_hbm.at[wid], dst_v)
        plsc.subcore_barrier()

        def gather(g, buf, sem):
            pltpu.async_copy(u_hbm.at[src_v.at[g]], rows_v.at[buf], sem)

        def gather_wait(g, buf, sem):
            pltpu.make_async_copy(u_hbm.at[src_v.at[g]],
                                  rows_v.at[buf], sem).wait()

        def scatter(g, buf, sem):
            pltpu.async_copy(rows_v.at[buf], acc.at[dst_v.at[g]],
                             sem, add=True)

        def scatter_wait(g, buf, sem):
            pltpu.make_async_copy(rows_v.at[buf], acc.at[dst_v.at[g]],
                                  sem).wait()

        gather(0, 0, gs0)

        @pl.loop(0, GPW // 2)
        def _(h):
            g0 = 2 * h
            g1 = g0 + 1
            gather_wait(g0, 0, gs0)

            @pl.when(h > 0)
            def _():
                scatter_wait(g0 - 1, 1, ss1)

            gather(g1, 1, gs1)
            scatter(g0, 0, ss0)
            gather_wait(g1, 1, gs1)
            scatter_wait(g0, 0, ss0)

            @pl.when(g0 + 2 < GPW)
            def _():
                gather(g0 + 2, 0, gs0)

            scatter(g1, 1, ss1)

        scatter_wait(GPW - 1, 1, ss1)
        plsc.subcore_barrier()
        pltpu.sync_copy(acc.at[pl.ds(base, rpt)],
                        out_hbm.at[c, pl.ds(base, rpt)])

    return hop_kernel(u, srcb, dstb, zrows)


def _tc_scale0(degp, xp, W48, NP, D, CP, n_real, BR):
    def body(d_ref, x_ref, w_ref, u_ref, dv_ref):
        i = pl.program_id(0)
        z = jnp.dot(x_ref[...], w_ref[...],
                    preferred_element_type=jnp.float32)
        deg = d_ref[0] + d_ref[1] + 1.0
        row = lax.broadcasted_iota(jnp.int32, (BR, 8), 0) + i * BR
        dinv = jnp.where(row < n_real, lax.rsqrt(deg), 0.0)
        dv_ref[...] = dinv
        u_ref[...] = z * dinv[:, 0:1]

    return pl.pallas_call(
        body,
        grid=(NP // BR,),
        in_specs=[pl.BlockSpec((2, BR, 8), lambda i: (0, i, 0)),
                  pl.BlockSpec((BR, D), lambda i: (i, 0)),
                  pl.BlockSpec((D, CP), lambda i: (0, 0))],
        out_specs=[pl.BlockSpec((BR, CP), lambda i: (i, 0)),
                   pl.BlockSpec((BR, 8), lambda i: (i, 0))],
        out_shape=[jax.ShapeDtypeStruct((NP, CP), jnp.float32),
                   jax.ShapeDtypeStruct((NP, 8), jnp.float32)],
    )(degp, xp, W48)


def _tc_scale1(hp, u0, dinv8, NP, CP, BR):
    def body(h_ref, u_ref, d_ref, o_ref):
        d = d_ref[:, 0:1]
        o_ref[...] = (h_ref[0] + h_ref[1] + u_ref[...]) * (d * d)

    return pl.pallas_call(
        body,
        grid=(NP // BR,),
        in_specs=[pl.BlockSpec((2, BR, CP), lambda i: (0, i, 0)),
                  pl.BlockSpec((BR, CP), lambda i: (i, 0)),
                  pl.BlockSpec((BR, 8), lambda i: (i, 0))],
        out_specs=pl.BlockSpec((BR, CP), lambda i: (i, 0)),
        out_shape=jax.ShapeDtypeStruct((NP, CP), jnp.float32),
    )(hp, u0, dinv8)


def _tc_out(qp, u1, dinv8, b48, NP, N, C, CP, BR):
    def body(q_ref, u_ref, d_ref, b_ref, o_ref):
        t = (q_ref[0] + q_ref[1] + u_ref[...]) * d_ref[:, 0:1]
        o40 = t[:, :C] + b_ref[0:1, :C]
        mx = jnp.max(o40, axis=1, keepdims=True)
        ex = jnp.exp(o40 - mx)
        sm = jnp.sum(ex, axis=1, keepdims=True)
        o_ref[...] = o40 - mx - jnp.log(sm)

    return pl.pallas_call(
        body,
        grid=(N // BR,),
        in_specs=[pl.BlockSpec((2, BR, CP), lambda i: (0, i, 0)),
                  pl.BlockSpec((BR, CP), lambda i: (i, 0)),
                  pl.BlockSpec((BR, 8), lambda i: (i, 0)),
                  pl.BlockSpec((8, CP), lambda i: (0, 0))],
        out_specs=pl.BlockSpec((BR, C), lambda i: (i, 0)),
        out_shape=jax.ShapeDtypeStruct((N, C), jnp.float32),
    )(qp, u1, dinv8, b48)


def kernel(x, edge_index, W, b):
    N, D = x.shape
    C = W.shape[1]
    E = edge_index.shape[1]
    CP = 48
    NP = -(-(N + 1) // (_NS * 8)) * (_NS * 8)
    GPW = -(-E // (_NW * _G))
    GPW += GPW % 2
    EP = GPW * _NW * _G

    src = edge_index[0]
    dst = edge_index[1]
    fill = N + jnp.arange(EP - E, dtype=jnp.int32) % (NP - N)
    srcb = jnp.concatenate([src, fill]).reshape(_NW, GPW, _G)
    dstb = jnp.concatenate([dst, fill]).reshape(_NW, GPW, _G)

    xp = jnp.pad(x, ((0, NP - N), (0, 0)))
    W48 = jnp.pad(W, ((0, 0), (0, CP - C)))
    b48 = jnp.broadcast_to(jnp.pad(b, (0, CP - C))[None, :], (8, CP))
    ones8 = jnp.ones((_G, 8), jnp.float32)
    z8 = jnp.zeros((NP // _NS, 8), jnp.float32)
    zCP = jnp.zeros((NP // _NS, CP), jnp.float32)

    BR = NP // 8
    degp = _sc_degree(dstb, ones8, z8, NP, GPW)
    u0, dinv8 = _tc_scale0(degp, xp, W48, NP, D, CP, N, BR)
    hp = _sc_hop(u0, srcb, dstb, zCP, NP, CP, GPW)
    u1 = _tc_scale1(hp, u0, dinv8, NP, CP, BR)
    qp = _sc_hop(u1, srcb, dstb, zCP, NP, CP, GPW)
    return _tc_out(qp, u1, dinv8, b48, NP, N, C, CP, 400)


# speedup vs baseline: 1.8089x; 1.0029x over previous
import functools

import jax
import jax.numpy as jnp
from jax import lax
from jax.experimental import pallas as pl
from jax.experimental.pallas import tpu as pltpu
from jax.experimental.pallas import tpu_sc as plsc

_NC = 2
_NS = 16
_NW = _NC * _NS
_G = 128


def _sc_degree(dstb, ones8, zrows8, NP, GPW):
    rpt = NP // _NS
    mesh = plsc.VectorSubcoreMesh(core_axis_name="c", subcore_axis_name="s")

    @functools.partial(
        pl.kernel,
        out_type=jax.ShapeDtypeStruct((_NC, NP, 8), jnp.float32),
        mesh=mesh,
        scratch_types=[
            pltpu.VMEM((GPW, _G), jnp.int32),
            pltpu.VMEM((_G, 8), jnp.float32),
            pltpu.VMEM_SHARED((NP, 8), jnp.float32),
        ],
        compiler_params=pltpu.CompilerParams(use_tc_tiling_on_sc=False),
    )
    def deg_kernel(dstb_hbm, ones_hbm, z_hbm, out_hbm, dst_v, ones_v, acc):
        c = lax.axis_index("c")
        s = lax.axis_index("s")
        wid = c * _NS + s
        base = s * rpt
        pltpu.sync_copy(z_hbm, acc.at[pl.ds(base, rpt)])
        pltpu.sync_copy(ones_hbm, ones_v)
        pltpu.sync_copy(dstb_hbm.at[wid], dst_v)
        plsc.subcore_barrier()

        @pl.loop(0, GPW)
        def _(g):
            pltpu.sync_copy(ones_v, acc.at[dst_v.at[g]], add=True)

        plsc.subcore_barrier()
        pltpu.sync_copy(acc.at[pl.ds(base, rpt)],
                        out_hbm.at[c, pl.ds(base, rpt)])

    return deg_kernel(dstb, ones8, zrows8)


def _sc_hop(u, srcb, dstb, zrows, NP, CP, GPW):
    rpt = NP // _NS
    mesh = plsc.VectorSubcoreMesh(core_axis_name="c", subcore_axis_name="s")

    @functools.partial(
        pl.kernel,
        out_type=jax.ShapeDtypeStruct((_NC, NP, CP), jnp.float32),
        mesh=mesh,
        scratch_types=[
            pltpu.VMEM((GPW, _G), jnp.int32),
            pltpu.VMEM((GPW, _G), jnp.int32),
            pltpu.VMEM((2, _G, CP), jnp.float32),
            pltpu.VMEM_SHARED((NP, CP), jnp.float32),
            pltpu.SemaphoreType.DMA,
            pltpu.SemaphoreType.DMA,
            pltpu.SemaphoreType.DMA,
            pltpu.SemaphoreType.DMA,
        ],
        compiler_params=pltpu.CompilerParams(use_tc_tiling_on_sc=False),
    )
    def hop_kernel(u_hbm, srcb_hbm, dstb_hbm, z_hbm, out_hbm,
                   src_v, dst_v, rows_v, acc, gs0, gs1, ss0, ss1):
        c = lax.axis_index("c")
        s = lax.axis_index("s")
        wid = c * _NS + s
        base = s * rpt
        pltpu.sync_copy(z_hbm, acc.at[pl.ds(base, rpt)])
        pltpu.sync_copy(srcb_hbm.at[wid], src_v)
        pltpu.sync_copy(dstb_hbm.at[wid], dst_v)
        plsc.subcore_barrier()

        def gather(g, buf, sem):
            pltpu.async_copy(u_hbm.at[src_v.at[g]], rows_v.at[buf], sem)

        def gather_wait(g, buf, sem):
            pltpu.make_async_copy(u_hbm.at[src_v.at[g]],
                                  rows_v.at[buf], sem).wait()

        def scatter(g, buf, sem):
            pltpu.async_copy(rows_v.at[buf], acc.at[dst_v.at[g]],
                             sem, add=True)

        def scatter_wait(g, buf, sem):
            pltpu.make_async_copy(rows_v.at[buf], acc.at[dst_v.at[g]],
                                  sem).wait()

        gather(0, 0, gs0)

        @pl.loop(0, GPW // 2)
        def _(h):
            g0 = 2 * h
            g1 = g0 + 1
            gather_wait(g0, 0, gs0)

            @pl.when(h > 0)
            def _():
                scatter_wait(g0 - 1, 1, ss1)

            gather(g1, 1, gs1)
            scatter(g0, 0, ss0)
            gather_wait(g1, 1, gs1)
            scatter_wait(g0, 0, ss0)

            @pl.when(g0 + 2 < GPW)
            def _():
                gather(g0 + 2, 0, gs0)

            scatter(g1, 1, ss1)

        scatter_wait(GPW - 1, 1, ss1)
        plsc.subcore_barrier()
        pltpu.sync_copy(acc.at[pl.ds(base, rpt)],
                        out_hbm.at[c, pl.ds(base, rpt)])

    return hop_kernel(u, srcb, dstb, zrows)


def _tc_scale0(degp, xp, W48, NP, D, CP, n_real, BR):
    def body(d_ref, x_ref, w_ref, u_ref, dv_ref):
        i = pl.program_id(0)
        z = jnp.dot(x_ref[...], w_ref[...],
                    preferred_element_type=jnp.float32)
        deg = d_ref[0] + d_ref[1] + 1.0
        row = lax.broadcasted_iota(jnp.int32, (BR, 8), 0) + i * BR
        dinv = jnp.where(row < n_real, lax.rsqrt(deg), 0.0)
        dv_ref[...] = dinv
        u_ref[...] = z * dinv[:, 0:1]

    return pl.pallas_call(
        body,
        grid=(NP // BR,),
        in_specs=[pl.BlockSpec((2, BR, 8), lambda i: (0, i, 0)),
                  pl.BlockSpec((BR, D), lambda i: (i, 0)),
                  pl.BlockSpec((D, CP), lambda i: (0, 0))],
        out_specs=[pl.BlockSpec((BR, CP), lambda i: (i, 0)),
                   pl.BlockSpec((BR, 8), lambda i: (i, 0))],
        out_shape=[jax.ShapeDtypeStruct((NP, CP), jnp.float32),
                   jax.ShapeDtypeStruct((NP, 8), jnp.float32)],
    )(degp, xp, W48)


def _tc_scale1(hp, u0, dinv8, NP, CP, BR):
    def body(h_ref, u_ref, d_ref, o_ref):
        d = d_ref[:, 0:1]
        o_ref[...] = (h_ref[0] + h_ref[1] + u_ref[...]) * (d * d)

    return pl.pallas_call(
        body,
        grid=(NP // BR,),
        in_specs=[pl.BlockSpec((2, BR, CP), lambda i: (0, i, 0)),
                  pl.BlockSpec((BR, CP), lambda i: (i, 0)),
                  pl.BlockSpec((BR, 8), lambda i: (i, 0))],
        out_specs=pl.BlockSpec((BR, CP), lambda i: (i, 0)),
        out_shape=jax.ShapeDtypeStruct((NP, CP), jnp.float32),
    )(hp, u0, dinv8)


def _tc_out(qp, u1, dinv8, b48, NP, N, C, CP, BR):
    def body(q_ref, u_ref, d_ref, b_ref, o_ref):
        t = (q_ref[0] + q_ref[1] + u_ref[...]) * d_ref[:, 0:1]
        o40 = t[:, :C] + b_ref[0:1, :C]
        mx = jnp.max(o40, axis=1, keepdims=True)
        ex = jnp.exp(o40 - mx)
        sm = jnp.sum(ex, axis=1, keepdims=True)
        o_ref[...] = o40 - mx - jnp.log(sm)

    return pl.pallas_call(
        body,
        grid=(N // BR,),
        in_specs=[pl.BlockSpec((2, BR, CP), lambda i: (0, i, 0)),
                  pl.BlockSpec((BR, CP), lambda i: (i, 0)),
                  pl.BlockSpec((BR, 8), lambda i: (i, 0)),
                  pl.BlockSpec((8, CP), lambda i: (0, 0))],
        out_specs=pl.BlockSpec((BR, C), lambda i: (i, 0)),
        out_shape=jax.ShapeDtypeStruct((N, C), jnp.float32),
    )(qp, u1, dinv8, b48)


def kernel(x, edge_index, W, b):
    N, D = x.shape
    C = W.shape[1]
    E = edge_index.shape[1]
    CP = -(-C // 8) * 8
    NP = -(-(N + 1) // (_NS * 8)) * (_NS * 8)
    GPW = -(-E // (_NW * _G))
    GPW += GPW % 2
    EP = GPW * _NW * _G

    src = edge_index[0]
    dst = edge_index[1]
    fill = N + jnp.arange(EP - E, dtype=jnp.int32) % (NP - N)
    srcb = jnp.concatenate([src, fill]).reshape(_NW, GPW, _G)
    dstb = jnp.concatenate([dst, fill]).reshape(_NW, GPW, _G)

    xp = jnp.pad(x, ((0, NP - N), (0, 0)))
    W48 = jnp.pad(W, ((0, 0), (0, CP - C)))
    b48 = jnp.broadcast_to(jnp.pad(b, (0, CP - C))[None, :], (8, CP))
    ones8 = jnp.ones((_G, 8), jnp.float32)
    z8 = jnp.zeros((NP // _NS, 8), jnp.float32)
    zCP = jnp.zeros((NP // _NS, CP), jnp.float32)

    BR = NP // 8
    degp = _sc_degree(dstb, ones8, z8, NP, GPW)
    u0, dinv8 = _tc_scale0(degp, xp, W48, NP, D, CP, N, BR)
    hp = _sc_hop(u0, srcb, dstb, zCP, NP, CP, GPW)
    u1 = _tc_scale1(hp, u0, dinv8, NP, CP, BR)
    qp = _sc_hop(u1, srcb, dstb, zCP, NP, CP, GPW)
    return _tc_out(qp, u1, dinv8, b48, NP, N, C, CP, 400)


# speedup vs baseline: 2.4819x; 1.3721x over previous
import functools

import jax
import jax.numpy as jnp
from jax import lax
from jax.experimental import pallas as pl
from jax.experimental.pallas import tpu as pltpu
from jax.experimental.pallas import tpu_sc as plsc

_NC = 2
_NS = 16
_NW = _NC * _NS
_G = 128


def _sc_degree(dstb, ones8, zrows8, NP, GPW):
    rpt = NP // _NS
    mesh = plsc.VectorSubcoreMesh(core_axis_name="c", subcore_axis_name="s")

    @functools.partial(
        pl.kernel,
        out_type=jax.ShapeDtypeStruct((_NC, NP, 8), jnp.float32),
        mesh=mesh,
        scratch_types=[
            pltpu.VMEM((GPW, _G), jnp.int32),
            pltpu.VMEM((_G, 8), jnp.float32),
            pltpu.VMEM_SHARED((NP, 8), jnp.float32),
            pltpu.SemaphoreType.DMA,
        ],
        compiler_params=pltpu.CompilerParams(use_tc_tiling_on_sc=False),
    )
    def deg_kernel(dstb_hbm, ones_hbm, z_hbm, out_hbm, dst_v, ones_v, acc,
                   sem):
        c = lax.axis_index("c")
        s = lax.axis_index("s")
        wid = c * _NS + s
        base = s * rpt
        pltpu.sync_copy(z_hbm, acc.at[pl.ds(base, rpt)])
        pltpu.sync_copy(ones_hbm, ones_v)
        pltpu.sync_copy(dstb_hbm.at[wid], dst_v)
        plsc.subcore_barrier()

        @pl.loop(0, GPW, step=8)
        def _(g0):
            for j in range(8):
                pltpu.async_copy(ones_v, acc.at[dst_v.at[g0 + j]], sem,
                                 add=True)
            for j in range(8):
                pltpu.make_async_copy(ones_v, acc.at[dst_v.at[g0 + j]],
                                      sem).wait()

        plsc.subcore_barrier()
        pltpu.sync_copy(acc.at[pl.ds(base, rpt)],
                        out_hbm.at[c, pl.ds(base, rpt)])

    return deg_kernel(dstb, ones8, zrows8)


def _sc_hop(u, srcb, dstb, zrows, NP, CP, GPW):
    rpt = NP // _NS
    mesh = plsc.VectorSubcoreMesh(core_axis_name="c", subcore_axis_name="s")

    @functools.partial(
        pl.kernel,
        out_type=jax.ShapeDtypeStruct((_NC, NP, CP), jnp.float32),
        mesh=mesh,
        scratch_types=[
            pltpu.VMEM((GPW, _G), jnp.int32),
            pltpu.VMEM((GPW, _G), jnp.int32),
            pltpu.VMEM((4, _G, CP), jnp.float32),
            pltpu.VMEM_SHARED((NP, CP), jnp.float32),
            [pltpu.SemaphoreType.DMA] * 4,
            [pltpu.SemaphoreType.DMA] * 4,
        ],
        compiler_params=pltpu.CompilerParams(use_tc_tiling_on_sc=False),
    )
    def hop_kernel(u_hbm, srcb_hbm, dstb_hbm, z_hbm, out_hbm,
                   src_v, dst_v, rows_v, acc, gs, ss):
        c = lax.axis_index("c")
        s = lax.axis_index("s")
        wid = c * _NS + s
        base = s * rpt
        pltpu.sync_copy(z_hbm, acc.at[pl.ds(base, rpt)])
        pltpu.sync_copy(srcb_hbm.at[wid], src_v)
        pltpu.sync_copy(dstb_hbm.at[wid], dst_v)
        plsc.subcore_barrier()

        def gather(g, buf, sem):
            pltpu.async_copy(u_hbm.at[src_v.at[g]], rows_v.at[buf], sem)

        def gather_wait(g, buf, sem):
            pltpu.make_async_copy(u_hbm.at[src_v.at[g]],
                                  rows_v.at[buf], sem).wait()

        def scatter(g, buf, sem):
            pltpu.async_copy(rows_v.at[buf], acc.at[dst_v.at[g]],
                             sem, add=True)

        def scatter_wait(g, buf, sem):
            pltpu.make_async_copy(rows_v.at[buf], acc.at[dst_v.at[g]],
                                  sem).wait()

        for j in range(4):
            gather(j, j, gs[j])

        @pl.loop(0, GPW // 4)
        def _(h):
            g0 = 4 * h
            for j in range(4):
                gather_wait(g0 + j, j, gs[j])
                scatter(g0 + j, j, ss[j])
            for j in range(4):
                @pl.when(g0 + 4 + j < GPW)
                def _(j=j):
                    scatter_wait(g0 + j, j, ss[j])
                    gather(g0 + 4 + j, j, gs[j])

        for j in range(4):
            scatter_wait(GPW - 4 + j, j, ss[j])
        plsc.subcore_barrier()
        pltpu.sync_copy(acc.at[pl.ds(base, rpt)],
                        out_hbm.at[c, pl.ds(base, rpt)])

    return hop_kernel(u, srcb, dstb, zrows)


def _tc_scale0(degp, xp, W48, NP, D, CP, n_real, BR):
    def body(d_ref, x_ref, w_ref, u_ref, dv_ref):
        i = pl.program_id(0)
        z = jnp.dot(x_ref[...], w_ref[...],
                    preferred_element_type=jnp.float32)
        deg = d_ref[0] + d_ref[1] + 1.0
        row = lax.broadcasted_iota(jnp.int32, (BR, 8), 0) + i * BR
        dinv = jnp.where(row < n_real, lax.rsqrt(deg), 0.0)
        dv_ref[...] = dinv
        u_ref[...] = z * dinv[:, 0:1]

    return pl.pallas_call(
        body,
        grid=(NP // BR,),
        in_specs=[pl.BlockSpec((2, BR, 8), lambda i: (0, i, 0)),
                  pl.BlockSpec((BR, D), lambda i: (i, 0)),
                  pl.BlockSpec((D, CP), lambda i: (0, 0))],
        out_specs=[pl.BlockSpec((BR, CP), lambda i: (i, 0)),
                   pl.BlockSpec((BR, 8), lambda i: (i, 0))],
        out_shape=[jax.ShapeDtypeStruct((NP, CP), jnp.float32),
                   jax.ShapeDtypeStruct((NP, 8), jnp.float32)],
    )(degp, xp, W48)


def _tc_scale1(hp, u0, dinv8, NP, CP, BR):
    def body(h_ref, u_ref, d_ref, o_ref):
        d = d_ref[:, 0:1]
        o_ref[...] = (h_ref[0] + h_ref[1] + u_ref[...]) * (d * d)

    return pl.pallas_call(
        body,
        grid=(NP // BR,),
        in_specs=[pl.BlockSpec((2, BR, CP), lambda i: (0, i, 0)),
                  pl.BlockSpec((BR, CP), lambda i: (i, 0)),
                  pl.BlockSpec((BR, 8), lambda i: (i, 0))],
        out_specs=pl.BlockSpec((BR, CP), lambda i: (i, 0)),
        out_shape=jax.ShapeDtypeStruct((NP, CP), jnp.float32),
    )(hp, u0, dinv8)


def _tc_out(qp, u1, dinv8, b48, NP, N, C, CP, BR):
    def body(q_ref, u_ref, d_ref, b_ref, o_ref):
        t = (q_ref[0] + q_ref[1] + u_ref[...]) * d_ref[:, 0:1]
        o40 = t[:, :C] + b_ref[0:1, :C]
        mx = jnp.max(o40, axis=1, keepdims=True)
        ex = jnp.exp(o40 - mx)
        sm = jnp.sum(ex, axis=1, keepdims=True)
        o_ref[...] = o40 - mx - jnp.log(sm)

    return pl.pallas_call(
        body,
        grid=(N // BR,),
        in_specs=[pl.BlockSpec((2, BR, CP), lambda i: (0, i, 0)),
                  pl.BlockSpec((BR, CP), lambda i: (i, 0)),
                  pl.BlockSpec((BR, 8), lambda i: (i, 0)),
                  pl.BlockSpec((8, CP), lambda i: (0, 0))],
        out_specs=pl.BlockSpec((BR, C), lambda i: (i, 0)),
        out_shape=jax.ShapeDtypeStruct((N, C), jnp.float32),
    )(qp, u1, dinv8, b48)


def kernel(x, edge_index, W, b):
    N, D = x.shape
    C = W.shape[1]
    E = edge_index.shape[1]
    CP = -(-C // 8) * 8
    NP = -(-(N + 1) // (_NS * 8)) * (_NS * 8)
    GPW = -(-E // (_NW * _G))
    GPW += (-GPW) % 8
    EP = GPW * _NW * _G

    src = edge_index[0]
    dst = edge_index[1]
    fill = N + jnp.arange(EP - E, dtype=jnp.int32) % (NP - N)
    srcb = jnp.concatenate([src, fill]).reshape(_NW, GPW, _G)
    dstb = jnp.concatenate([dst, fill]).reshape(_NW, GPW, _G)

    xp = jnp.pad(x, ((0, NP - N), (0, 0)))
    W48 = jnp.pad(W, ((0, 0), (0, CP - C)))
    b48 = jnp.broadcast_to(jnp.pad(b, (0, CP - C))[None, :], (8, CP))
    ones8 = jnp.ones((_G, 8), jnp.float32)
    z8 = jnp.zeros((NP // _NS, 8), jnp.float32)
    zCP = jnp.zeros((NP // _NS, CP), jnp.float32)

    BR = NP // 8
    degp = _sc_degree(dstb, ones8, z8, NP, GPW)
    u0, dinv8 = _tc_scale0(degp, xp, W48, NP, D, CP, N, BR)
    hp = _sc_hop(u0, srcb, dstb, zCP, NP, CP, GPW)
    u1 = _tc_scale1(hp, u0, dinv8, NP, CP, BR)
    qp = _sc_hop(u1, srcb, dstb, zCP, NP, CP, GPW)
    return _tc_out(qp, u1, dinv8, b48, NP, N, C, CP, 400)
